# baseline (speedup 1.0000x reference)
import functools

import jax
import jax.numpy as jnp
from jax.experimental import pallas as pl
from jax.experimental.pallas import tpu as pltpu
from jax.experimental.pallas import tpu_sc as plsc

DIM = 64
B = 16384

_W_ENT = 256
_W_REL = 128


def _fuse_kernel(rel_ref, nrm_ref, w_ref):
    n = nrm_ref[...]
    nn = n / jnp.maximum(
        jnp.sqrt(jnp.sum(n * n, axis=-1, keepdims=True)), 1e-12)
    w_ref[...] = jnp.concatenate([rel_ref[...], nn], axis=-1)


def _fused_rel_table(rel_emb, norm_vec):
    r = rel_emb.shape[0]
    return pl.pallas_call(
        _fuse_kernel,
        out_shape=jax.ShapeDtypeStruct((r, 2 * DIM), jnp.float32),
    )(rel_emb, norm_vec)


_TCOLS = 32768
_HALF = _TCOLS // 2


def _transpose_kernel(ent_t_ref, out_ref):
    out_ref[:, :DIM] = ent_t_ref[:, :_HALF].T
    out_ref[:, DIM:] = ent_t_ref[:, _HALF:].T


def _transpose_pair(ent_t):
    ent = ent_t.shape[1]
    nblk = pl.cdiv(ent, _TCOLS)
    return pl.pallas_call(
        _transpose_kernel,
        grid=(nblk,),
        in_specs=[pl.BlockSpec((DIM, _TCOLS), lambda i: (0, i))],
        out_specs=pl.BlockSpec((_HALF, 2 * DIM), lambda i: (i, 0)),
        out_shape=jax.ShapeDtypeStruct((nblk * _HALF, 2 * DIM), jnp.float32),
    )(ent_t)


def _sc_gather_rows(table, idx, window):
    n = idx.shape[1]
    lanes = 2 * DIM
    mesh = plsc.VectorSubcoreMesh(core_axis_name="c", subcore_axis_name="s")

    @functools.partial(
        pl.kernel,
        out_type=jax.ShapeDtypeStruct((n, lanes), jnp.float32),
        mesh=mesh,
    )
    def sc_kernel(t_hbm, i_hbm, o_hbm):
        def body(i_vmem, o_vmem):
            pltpu.sync_copy(t_hbm.at[i_vmem.at[0]], o_vmem)

        pltpu.emit_pipeline(
            body,
            grid=(n // window,),
            in_specs=[pl.BlockSpec((1, window), lambda i: (0, i))],
            out_specs=[pl.BlockSpec((window, lanes), lambda i: (i, 0))],
            core_axis_name=("c", "s"),
            dimension_semantics=(pltpu.PARALLEL,),
        )(i_hbm, o_hbm)

    return sc_kernel(table, idx)


_BLK = 2048


def _tc_loss_kernel(ph, pt, nh, nt, pw, nw,
                    ps, pts, ns, nts, out_ref):
    def pick(full, s8):
        s = s8[...].T[:, 0:1]
        return jnp.where(s > 0.5, full[:, DIM:], full[:, :DIM])

    def score(h2, hs, t2, ts, w):
        r = w[:, :DIM]
        n = w[:, DIM:]
        h = pick(h2, hs)
        t = pick(t2, ts)

        def transfer(e):
            return e - jnp.sum(e * n, axis=-1, keepdims=True) * n

        d = transfer(h) + r - transfer(t)
        return jnp.sqrt(jnp.sum(d * d, axis=-1))

    p_score = score(ph[...], ps[...], pt[...], pts[...], pw[...])
    n_score = score(nh[...], ns[...], nt[...], nts[...], nw[...])
    partial = jnp.sum(jnp.maximum(0.0, p_score - n_score + 1.0))

    @pl.when(pl.program_id(0) == 0)
    def _():
        out_ref[0, 0] = 0.0

    out_ref[0, 0] += partial


def _tc_loss(ent_rows, w_rows, sel_ent, bsz, woff):
    nb = bsz // _BLK
    wb = woff // _BLK
    full = lambda off: pl.BlockSpec((_BLK, 2 * DIM),
                                    lambda i, o=off: (i + o, 0))
    sel = lambda off: pl.BlockSpec((8, _BLK), lambda i, o=off: (0, i + o))
    return pl.pallas_call(
        _tc_loss_kernel,
        grid=(nb,),
        in_specs=[
            full(0), full(nb), full(2 * nb), full(3 * nb),
            full(wb), full(wb + B // _BLK),
            sel(0), sel(nb), sel(2 * nb), sel(3 * nb),
        ],
        out_specs=pl.BlockSpec(memory_space=pltpu.SMEM),
        out_shape=jax.ShapeDtypeStruct((1, 1), jnp.float32),
    )(ent_rows, ent_rows, ent_rows, ent_rows, w_rows, w_rows,
      sel_ent, sel_ent, sel_ent, sel_ent)


_NCHUNK = 1


def kernel(ent_emb, rel_emb, norm_vec, pos_h, pos_r, pos_t, neg_h, neg_r, neg_t):
    w_table = _fused_rel_table(rel_emb, norm_vec)
    idx_rel = jnp.concatenate([pos_r, neg_r])
    n_rel = idx_rel.shape[0]
    idx_rel_r = idx_rel.reshape(1, n_rel)
    w_rows = _sc_gather_rows(w_table, idx_rel_r, _W_REL)
    ent_t = jax.lax.optimization_barrier(
        (ent_emb.T, w_table, idx_rel_r))[0]

    ent_pairs = _transpose_pair(ent_t)

    bc = B // _NCHUNK
    loss = jnp.float32(0)
    for c in range(_NCHUNK):
        sl = slice(c * bc, (c + 1) * bc)
        idx_c = jnp.concatenate([pos_h[sl], pos_t[sl], neg_h[sl], neg_t[sl]])
        n_c = idx_c.shape[0]
        row_c = (idx_c // _TCOLS) * _HALF + idx_c % _HALF
        rows_c = _sc_gather_rows(ent_pairs, row_c.reshape(1, n_c), _W_ENT)
        sel_c = jnp.broadcast_to(
            ((idx_c // _HALF) % 2).astype(jnp.float32)[None, :], (8, n_c))
        loss = loss + _tc_loss(rows_c, w_rows, sel_c, bc, c * bc)[0, 0]
    return loss


# speedup vs baseline: 1.0132x; 1.0132x over previous
import functools

import jax
import jax.numpy as jnp
from jax.experimental import pallas as pl
from jax.experimental.pallas import tpu as pltpu
from jax.experimental.pallas import tpu_sc as plsc

DIM = 64
B = 16384

_W_ENT = 256
_W_REL = 128


def _fuse_kernel(rel_ref, nrm_ref, w_ref):
    n = nrm_ref[...]
    nn = n / jnp.maximum(
        jnp.sqrt(jnp.sum(n * n, axis=-1, keepdims=True)), 1e-12)
    w_ref[...] = jnp.concatenate([rel_ref[...], nn], axis=-1)


def _fused_rel_table(rel_emb, norm_vec):
    r = rel_emb.shape[0]
    return pl.pallas_call(
        _fuse_kernel,
        out_shape=jax.ShapeDtypeStruct((r, 2 * DIM), jnp.float32),
    )(rel_emb, norm_vec)


_TCOLS = 32768
_HALF = _TCOLS // 2


def _transpose_kernel(ent_t_ref, out_ref):
    x = ent_t_ref[...]
    out_ref[...] = jnp.concatenate(
        [x[:, :_HALF].T, x[:, _HALF:].T], axis=-1)


def _transpose_pair(ent_t):
    ent = ent_t.shape[1]
    nblk = pl.cdiv(ent, _TCOLS)
    return pl.pallas_call(
        _transpose_kernel,
        grid=(nblk,),
        in_specs=[pl.BlockSpec((DIM, _TCOLS), lambda i: (0, i))],
        out_specs=pl.BlockSpec((_HALF, 2 * DIM), lambda i: (i, 0)),
        out_shape=jax.ShapeDtypeStruct((nblk * _HALF, 2 * DIM), jnp.float32),
    )(ent_t)


def _sc_gather_rows(table, idx, window):
    n = idx.shape[1]
    lanes = 2 * DIM
    mesh = plsc.VectorSubcoreMesh(core_axis_name="c", subcore_axis_name="s")

    @functools.partial(
        pl.kernel,
        out_type=jax.ShapeDtypeStruct((n, lanes), jnp.float32),
        mesh=mesh,
    )
    def sc_kernel(t_hbm, i_hbm, o_hbm):
        def body(i_vmem, o_vmem):
            pltpu.sync_copy(t_hbm.at[i_vmem.at[0]], o_vmem)

        pltpu.emit_pipeline(
            body,
            grid=(n // window,),
            in_specs=[pl.BlockSpec((1, window), lambda i: (0, i))],
            out_specs=[pl.BlockSpec((window, lanes), lambda i: (i, 0))],
            core_axis_name=("c", "s"),
            dimension_semantics=(pltpu.PARALLEL,),
        )(i_hbm, o_hbm)

    return sc_kernel(table, idx)


_BLK = 2048


def _tc_loss_kernel(ph, pt, nh, nt, pw, nw,
                    ps, pts, ns, nts, out_ref):
    def pick(full, s8):
        s = s8[...].T[:, 0:1]
        return jnp.where(s > 0.5, full[:, DIM:], full[:, :DIM])

    def score(h2, hs, t2, ts, w):
        r = w[:, :DIM]
        n = w[:, DIM:]
        h = pick(h2, hs)
        t = pick(t2, ts)

        def transfer(e):
            return e - jnp.sum(e * n, axis=-1, keepdims=True) * n

        d = transfer(h) + r - transfer(t)
        return jnp.sqrt(jnp.sum(d * d, axis=-1))

    p_score = score(ph[...], ps[...], pt[...], pts[...], pw[...])
    n_score = score(nh[...], ns[...], nt[...], nts[...], nw[...])
    partial = jnp.sum(jnp.maximum(0.0, p_score - n_score + 1.0))

    @pl.when(pl.program_id(0) == 0)
    def _():
        out_ref[0, 0] = 0.0

    out_ref[0, 0] += partial


def _tc_loss(ent_rows, w_rows, sel_ent, bsz, woff):
    nb = bsz // _BLK
    wb = woff // _BLK
    full = lambda off: pl.BlockSpec((_BLK, 2 * DIM),
                                    lambda i, o=off: (i + o, 0))
    sel = lambda off: pl.BlockSpec((8, _BLK), lambda i, o=off: (0, i + o))
    return pl.pallas_call(
        _tc_loss_kernel,
        grid=(nb,),
        in_specs=[
            full(0), full(nb), full(2 * nb), full(3 * nb),
            full(wb), full(wb + B // _BLK),
            sel(0), sel(nb), sel(2 * nb), sel(3 * nb),
        ],
        out_specs=pl.BlockSpec(memory_space=pltpu.SMEM),
        out_shape=jax.ShapeDtypeStruct((1, 1), jnp.float32),
    )(ent_rows, ent_rows, ent_rows, ent_rows, w_rows, w_rows,
      sel_ent, sel_ent, sel_ent, sel_ent)


_NCHUNK = 1


def kernel(ent_emb, rel_emb, norm_vec, pos_h, pos_r, pos_t, neg_h, neg_r, neg_t):
    w_table = _fused_rel_table(rel_emb, norm_vec)
    idx_rel = jnp.concatenate([pos_r, neg_r])
    n_rel = idx_rel.shape[0]
    idx_rel_r = idx_rel.reshape(1, n_rel)
    w_rows = _sc_gather_rows(w_table, idx_rel_r, _W_REL)

    ent_pairs = _transpose_pair(ent_emb.T)

    bc = B // _NCHUNK
    loss = jnp.float32(0)
    for c in range(_NCHUNK):
        sl = slice(c * bc, (c + 1) * bc)
        idx_c = jnp.concatenate([pos_h[sl], pos_t[sl], neg_h[sl], neg_t[sl]])
        n_c = idx_c.shape[0]
        row_c = (idx_c // _TCOLS) * _HALF + idx_c % _HALF
        rows_c = _sc_gather_rows(ent_pairs, row_c.reshape(1, n_c), _W_ENT)
        sel_c = jnp.broadcast_to(
            ((idx_c // _HALF) % 2).astype(jnp.float32)[None, :], (8, n_c))
        loss = loss + _tc_loss(rows_c, w_rows, sel_c, bc, c * bc)[0, 0]
    return loss
